# baseline (speedup 1.0000x reference)
import functools

import jax
import jax.numpy as jnp
from jax import lax
from jax.experimental import pallas as pl
from jax.experimental.pallas import tpu as pltpu
from jax.experimental.pallas import tpu_sc as plsc

VOCAB = 100000
BATCH = 64
ROW = 3 * VOCAB
NC, NS, L = 2, 16, 16
NW = NC * NS
ROWS_PER_W = BATCH // NW
CHUNK = 30000
NCHUNK = ROW // CHUNK
VECS = CHUNK // L
NEG = float(jnp.finfo(jnp.float32).min)
IMAX = 2**31 - 1

_mesh = plsc.VectorSubcoreMesh(
    core_axis_name="c", subcore_axis_name="s", num_cores=NC, num_subcores=NS)


@functools.partial(
    pl.kernel,
    out_type=jax.ShapeDtypeStruct((NW, L), jnp.int32),
    mesh=_mesh,
    scratch_types=[
        pltpu.VMEM((CHUNK,), jnp.float32),
        pltpu.VMEM((CHUNK,), jnp.float32),
        pltpu.VMEM((L,), jnp.int32),
        pltpu.VMEM((L,), jnp.int32),
        pltpu.SemaphoreType.DMA,
        pltpu.SemaphoreType.DMA,
    ],
)
def _topk_sc(in_hbm, base_hbm, out_hbm, buf0, buf1, basev, resv, sem0, sem1):
    wid = lax.axis_index("s") * NC + lax.axis_index("c")
    lanes = lax.iota(jnp.int32, L)
    res = jnp.zeros((L,), jnp.int32)
    bufs = (buf0, buf1)
    sems = (sem0, sem1)
    for p in range(ROWS_PER_W):
        r = wid * ROWS_PER_W + p
        rbase = r * ROW
        pltpu.sync_copy(base_hbm.at[r], basev)
        base_vec = basev[...]
        cps = [pltpu.async_copy(in_hbm.at[pl.ds(rbase, CHUNK)], buf0, sem0)]
        m0 = jnp.full((L,), NEG, jnp.float32)
        m1 = m0
        m2 = m0
        i0 = jnp.zeros((L,), jnp.int32)
        i1 = i0
        i2 = i0
        vi = lanes
        for c in range(NCHUNK):
            if c + 1 < NCHUNK:
                cps.append(pltpu.async_copy(
                    in_hbm.at[pl.ds(rbase + (c + 1) * CHUNK, CHUNK)],
                    bufs[(c + 1) % 2], sems[(c + 1) % 2]))
            cps[c].wait()
            buf = bufs[c % 2]

            def body(i, carry, buf=buf):
                m0, m1, m2, i0, i1, i2, vi = carry
                v = buf[pl.ds(i * L, L)]
                c0 = v > m0
                c1 = v > m1
                c2 = v > m2
                nm2 = jnp.where(c1, m1, jnp.where(c2, v, m2))
                ni2 = jnp.where(c1, i1, jnp.where(c2, vi, i2))
                nm1 = jnp.where(c0, m0, jnp.where(c1, v, m1))
                ni1 = jnp.where(c0, i0, jnp.where(c1, vi, i1))
                nm0 = jnp.where(c0, v, m0)
                ni0 = jnp.where(c0, vi, i0)
                return (nm0, nm1, nm2, ni0, ni1, ni2, vi + L)

            m0, m1, m2, i0, i1, i2, vi = lax.fori_loop(
                0, VECS, body, (m0, m1, m2, i0, i1, i2, vi))
        def allreduce(v, op):
            for s in (1, 2, 4, 8):
                perm = jnp.bitwise_xor(lanes, s)
                v = op(v, v.at[perm].get(mode="promise_in_bounds"))
            return v

        zero = jnp.zeros((L,), jnp.int32)
        for k in range(3):
            cur = allreduce(m0, jnp.maximum)
            cand = jnp.where(m0 == cur, i0, IMAX)
            widx = allreduce(cand, jnp.minimum)
            isw = (m0 == cur) & (i0 == widx)
            q = (jnp.where(widx >= VOCAB, 1, zero)
                 + jnp.where(widx >= 2 * VOCAB, 1, zero))
            tok = widx - q * VOCAB
            beam = q + base_vec
            res = jnp.where(lanes == 6 * p + k, tok, res)
            res = jnp.where(lanes == 6 * p + 3 + k, beam, res)
            m0 = jnp.where(isw, m1, m0)
            i0 = jnp.where(isw, i1, i0)
            m1 = jnp.where(isw, m2, m1)
            i1 = jnp.where(isw, i2, i1)
            m2 = jnp.where(isw, NEG, m2)
    resv[...] = res
    pltpu.sync_copy(resv, out_hbm.at[wid])


def kernel(input, index, cur_beam_size):
    cbs = jnp.asarray(cur_beam_size, jnp.int32)
    base = (jnp.arange(BATCH, dtype=jnp.int32) * cbs)[:, None]
    base_mat = jnp.broadcast_to(base, (BATCH, L))
    out = _topk_sc(input.reshape(-1), base_mat)
    x = out[:, :12].reshape(NW, ROWS_PER_W, 2, 3)
    toks = x[:, :, 0, :].reshape(1, BATCH * 3)
    beams = x[:, :, 1, :].reshape(BATCH * 3)
    return toks, beams


# speedup vs baseline: 1.0892x; 1.0892x over previous
import functools

import jax
import jax.numpy as jnp
from jax import lax
from jax.experimental import pallas as pl
from jax.experimental.pallas import tpu as pltpu
from jax.experimental.pallas import tpu_sc as plsc

VOCAB = 100000
BATCH = 64
ROW = 3 * VOCAB
NC, NS, L = 2, 16, 16
NW = NC * NS
ROWS_PER_W = BATCH // NW
CHUNK = 30000
NCHUNK = ROW // CHUNK
BLK = 1200
BV = BLK // L
NB = ROW // BLK
BPC = CHUNK // BLK
NEG = float(jnp.finfo(jnp.float32).min)
IMAX = 2**31 - 1

_mesh = plsc.VectorSubcoreMesh(
    core_axis_name="c", subcore_axis_name="s", num_cores=NC, num_subcores=NS)


@functools.partial(
    pl.kernel,
    out_type=jax.ShapeDtypeStruct((NW, L), jnp.int32),
    mesh=_mesh,
    scratch_types=[
        pltpu.VMEM((CHUNK,), jnp.float32),
        pltpu.VMEM((CHUNK,), jnp.float32),
        pltpu.VMEM((3 * BLK,), jnp.float32),
        pltpu.VMEM((NB * L,), jnp.float32),
        pltpu.VMEM((L,), jnp.int32),
        pltpu.VMEM((L,), jnp.int32),
        pltpu.SemaphoreType.DMA,
        pltpu.SemaphoreType.DMA,
    ],
)
def _topk_sc(in_hbm, base_hbm, out_hbm, buf0, buf1, cbuf, bmax, basev, resv,
             sem0, sem1):
    wid = lax.axis_index("s") * NC + lax.axis_index("c")
    lanes = lax.iota(jnp.int32, L)
    zero = jnp.zeros((L,), jnp.int32)
    res = zero
    bufs = (buf0, buf1)
    sems = (sem0, sem1)

    def allreduce(v, op):
        for s in (1, 2, 4, 8):
            perm = jnp.bitwise_xor(lanes, s)
            v = op(v, v.at[perm].get(mode="promise_in_bounds"))
        return v

    for p in range(ROWS_PER_W):
        r = wid * ROWS_PER_W + p
        rbase = r * ROW
        pltpu.sync_copy(base_hbm.at[r], basev)
        base_vec = basev[...]

        cps = [pltpu.async_copy(in_hbm.at[pl.ds(rbase, CHUNK)], buf0, sem0)]
        for c in range(NCHUNK):
            if c + 1 < NCHUNK:
                cps.append(pltpu.async_copy(
                    in_hbm.at[pl.ds(rbase + (c + 1) * CHUNK, CHUNK)],
                    bufs[(c + 1) % 2], sems[(c + 1) % 2]))
            cps[c].wait()
            buf = bufs[c % 2]

            def blk_body(b, carry, buf=buf, coff=c * BPC):
                off = b * BLK
                vs = [buf[pl.ds(off + j * L, L)] for j in range(BV)]
                while len(vs) > 1:
                    nxt = [jnp.maximum(vs[t], vs[t + 1])
                           for t in range(0, len(vs) - 1, 2)]
                    if len(vs) % 2:
                        nxt.append(vs[-1])
                    vs = nxt
                bmax[pl.ds((coff + b) * L, L)] = vs[0]
                return carry

            lax.fori_loop(0, BPC, blk_body, 0)

        m0 = jnp.full((L,), NEG, jnp.float32)
        m1 = m0
        m2 = m0
        b0 = zero
        b1 = zero
        b2 = zero

        def ins_body(blk, carry):
            m0, m1, m2, b0, b1, b2 = carry
            bm = bmax[pl.ds(blk * L, L)]
            bv = zero + blk
            c0 = bm > m0
            c1 = bm > m1
            c2 = bm > m2
            nm2 = jnp.where(c1, m1, jnp.where(c2, bm, m2))
            nb2 = jnp.where(c1, b1, jnp.where(c2, bv, b2))
            nm1 = jnp.where(c0, m0, jnp.where(c1, bm, m1))
            nb1 = jnp.where(c0, b0, jnp.where(c1, bv, b1))
            nm0 = jnp.where(c0, bm, m0)
            nb0 = jnp.where(c0, bv, b0)
            return (nm0, nm1, nm2, nb0, nb1, nb2)

        m0, m1, m2, b0, b1, b2 = lax.fori_loop(
            0, NB, ins_body, (m0, m1, m2, b0, b1, b2))

        wbs = []
        for k in range(3):
            cur = allreduce(m0, jnp.maximum)
            cand = jnp.minimum(
                jnp.where(m0 == cur, b0, IMAX),
                jnp.minimum(jnp.where(m1 == cur, b1, IMAX),
                            jnp.where(m2 == cur, b2, IMAX)))
            wb = allreduce(cand, jnp.minimum)
            wbs.append(wb)
            t0 = b0 == wb
            m0 = jnp.where(t0, m1, m0)
            b0 = jnp.where(t0, b1, b0)
            t1 = t0 | (b1 == wb)
            m1 = jnp.where(t1, m2, m1)
            b1 = jnp.where(t1, b2, b1)
            t2 = t1 | (b2 == wb)
            m2 = jnp.where(t2, NEG, m2)

        w0, w1, w2 = wbs
        lo01 = jnp.minimum(w0, w1)
        hi01 = jnp.maximum(w0, w1)
        s0 = jnp.minimum(lo01, w2)
        s2 = jnp.maximum(hi01, w2)
        s1 = (w0 + w1 + w2) - s0 - s2
        ks = [s0[0], s1[0], s2[0]]

        ccps = [pltpu.async_copy(
            in_hbm.at[pl.ds(rbase + ks[t] * BLK, BLK)],
            cbuf.at[pl.ds(t * BLK, BLK)], sem0) for t in range(3)]
        m0 = jnp.full((L,), NEG, jnp.float32)
        m1 = m0
        m2 = m0
        i0 = zero
        i1 = zero
        i2 = zero
        for t in range(3):
            ccps[t].wait()
            bvec = zero + ks[t] * BLK + lanes

            def scan_body(j, carry, t=t, bvec=bvec):
                m0, m1, m2, i0, i1, i2 = carry
                v = cbuf[pl.ds(t * BLK + j * L, L)]
                vi = bvec + j * L
                c0 = v > m0
                c1 = v > m1
                c2 = v > m2
                nm2 = jnp.where(c1, m1, jnp.where(c2, v, m2))
                ni2 = jnp.where(c1, i1, jnp.where(c2, vi, i2))
                nm1 = jnp.where(c0, m0, jnp.where(c1, v, m1))
                ni1 = jnp.where(c0, i0, jnp.where(c1, vi, i1))
                nm0 = jnp.where(c0, v, m0)
                ni0 = jnp.where(c0, vi, i0)
                return (nm0, nm1, nm2, ni0, ni1, ni2)

            m0, m1, m2, i0, i1, i2 = lax.fori_loop(
                0, BV, scan_body, (m0, m1, m2, i0, i1, i2))

        for k in range(3):
            cur = allreduce(m0, jnp.maximum)
            cand = jnp.where(m0 == cur, i0, IMAX)
            widx = allreduce(cand, jnp.minimum)
            isw = (m0 == cur) & (i0 == widx)
            q = (jnp.where(widx >= VOCAB, 1, zero)
                 + jnp.where(widx >= 2 * VOCAB, 1, zero))
            tok = widx - q * VOCAB
            beam = q + base_vec
            res = jnp.where(lanes == 6 * p + k, tok, res)
            res = jnp.where(lanes == 6 * p + 3 + k, beam, res)
            m0 = jnp.where(isw, m1, m0)
            i0 = jnp.where(isw, i1, i0)
            m1 = jnp.where(isw, m2, m1)
            i1 = jnp.where(isw, i2, i1)
            m2 = jnp.where(isw, NEG, m2)

    resv[...] = res
    pltpu.sync_copy(resv, out_hbm.at[wid])


def kernel(input, index, cur_beam_size):
    cbs = jnp.asarray(cur_beam_size, jnp.int32)
    base = (jnp.arange(BATCH, dtype=jnp.int32) * cbs)[:, None]
    base_mat = jnp.broadcast_to(base, (BATCH, L))
    out = _topk_sc(input.reshape(-1), base_mat)
    x = out[:, :12].reshape(NW, ROWS_PER_W, 2, 3)
    toks = x[:, :, 0, :].reshape(1, BATCH * 3)
    beams = x[:, :, 1, :].reshape(BATCH * 3)
    return toks, beams


# speedup vs baseline: 12.4833x; 11.4606x over previous
import functools

import jax
import jax.numpy as jnp
from jax import lax
from jax.experimental import pallas as pl
from jax.experimental.pallas import tpu as pltpu
from jax.experimental.pallas import tpu_sc as plsc

VOCAB = 100000
BATCH = 64
ROW = 3 * VOCAB
NC, NS, L = 2, 16, 16
NT = 2344
QT = 586
QCOLS = QT * 128
CT = 32
CCOLS = CT * 128
CSTARTS = [i * CT for i in range(18)] + [QT - CT]
BCOLS = 256
NBQ = QCOLS // BCOLS
BV = BCOLS // L
NEG = float(jnp.finfo(jnp.float32).min)
IMAX = 2**31 - 1

_mesh = plsc.VectorSubcoreMesh(
    core_axis_name="c", subcore_axis_name="s", num_cores=NC, num_subcores=NS)


@functools.partial(
    pl.kernel,
    out_type=jax.ShapeDtypeStruct((NC * NS * L,), jnp.int32),
    mesh=_mesh,
    compiler_params=pltpu.CompilerParams(use_tc_tiling_on_sc=True),
    scratch_types=[
        pltpu.VMEM((8, CCOLS), jnp.float32),
        pltpu.VMEM((8, CCOLS), jnp.float32),
        pltpu.VMEM((8 * NBQ * L,), jnp.float32),
        pltpu.VMEM((24, BCOLS), jnp.float32),
        pltpu.VMEM((160,), jnp.float32),
        pltpu.VMEM((160,), jnp.int32),
        pltpu.VMEM((32,), jnp.float32),
        pltpu.VMEM((32,), jnp.int32),
        pltpu.VMEM((L,), jnp.int32),
        pltpu.VMEM((L,), jnp.int32),
        pltpu.VMEM_SHARED((NS * 32,), jnp.float32),
        pltpu.VMEM_SHARED((NS * 32,), jnp.int32),
        pltpu.SemaphoreType.DMA,
        pltpu.SemaphoreType.DMA,
    ],
)
def _topk_sc(in_hbm, cbs_hbm, out_hbm, buf0, buf1, bmax, cbuf, mval, midx,
             stgv, stgi, cbsv, resv, sval_sh, sidx_sh, sem0, sem1):
    c = lax.axis_index("c")
    s = lax.axis_index("s")
    g = c * 4 + lax.shift_right_logical(s, 2)
    q = lax.bitwise_and(s, 3)
    row0 = pl.multiple_of(g * 8, 8)
    qcol0 = q * QCOLS
    lanes = lax.iota(jnp.int32, L)
    zero = jnp.zeros((L,), jnp.int32)
    negv = jnp.full((L,), NEG, jnp.float32)
    qcol0v = zero + qcol0
    cbs_vec = None
    pltpu.sync_copy(cbs_hbm, cbsv)
    cbs_vec = cbsv[...]
    bufs = (buf0, buf1)
    sems = (sem0, sem1)

    def allreduce(v, op):
        for si in (1, 2, 4, 8):
            perm = jnp.bitwise_xor(lanes, si)
            v = op(v, v.at[perm].get(mode="promise_in_bounds"))
        return v

    def src_slab(ci):
        coff = pl.multiple_of(qcol0 + CSTARTS[ci] * 128, 128)
        return in_hbm.at[pl.ds(row0, 8), pl.ds(coff, CCOLS)]

    cps = [pltpu.async_copy(src_slab(0), buf0, sem0)]
    for ci in range(len(CSTARTS)):
        if ci + 1 < len(CSTARTS):
            cps.append(pltpu.async_copy(
                src_slab(ci + 1), bufs[(ci + 1) % 2], sems[(ci + 1) % 2]))
        cps[ci].wait()
        buf = bufs[ci % 2]
        cblk0 = CSTARTS[ci] // 2
        last = ci == len(CSTARTS) - 1

        def blk_body(ii, carry, buf=buf, cblk0=cblk0, last=last):
            sl = lax.shift_right_logical(ii, 4)
            b = lax.bitwise_and(ii, 15)
            vs = []
            for j in range(BV):
                v = buf[sl, pl.ds(b * BCOLS + j * L, L)]
                if last:
                    colv = qcol0v + (cblk0 + b) * BCOLS + j * L + lanes
                    v = jnp.where(colv < ROW, v, negv)
                vs.append(v)
            while len(vs) > 1:
                nxt = [jnp.maximum(vs[t], vs[t + 1])
                       for t in range(0, len(vs) - 1, 2)]
                if len(vs) % 2:
                    nxt.append(vs[-1])
                vs = nxt
            bmax[pl.ds((sl * NBQ + cblk0 + b) * L, L)] = vs[0]
            return carry

        lax.fori_loop(0, 8 * (CT // 2), blk_body, 0)

    def row_body(sl, rcarry):
        sv0, sv1, si0, si1 = rcarry
        m0 = negv
        m1 = negv
        m2 = negv
        b0 = zero
        b1 = zero
        b2 = zero

        def ins_body(blk, carry):
            m0, m1, m2, b0, b1, b2 = carry
            bm = bmax[pl.ds((sl * NBQ + blk) * L, L)]
            bv = zero + blk
            c0 = bm > m0
            c1 = bm > m1
            c2 = bm > m2
            nm2 = jnp.where(c1, m1, jnp.where(c2, bm, m2))
            nb2 = jnp.where(c1, b1, jnp.where(c2, bv, b2))
            nm1 = jnp.where(c0, m0, jnp.where(c1, bm, m1))
            nb1 = jnp.where(c0, b0, jnp.where(c1, bv, b1))
            nm0 = jnp.where(c0, bm, m0)
            nb0 = jnp.where(c0, bv, b0)
            return (nm0, nm1, nm2, nb0, nb1, nb2)

        m0, m1, m2, b0, b1, b2 = lax.fori_loop(
            0, NBQ, ins_body, (m0, m1, m2, b0, b1, b2))

        wbs = []
        for k in range(3):
            cur = allreduce(m0, jnp.maximum)
            cand = jnp.minimum(
                jnp.where(m0 == cur, b0, IMAX),
                jnp.minimum(jnp.where(m1 == cur, b1, IMAX),
                            jnp.where(m2 == cur, b2, IMAX)))
            wb = allreduce(cand, jnp.minimum)
            wbs.append(wb)
            t0 = b0 == wb
            m0 = jnp.where(t0, m1, m0)
            b0 = jnp.where(t0, b1, b0)
            t1 = t0 | (b1 == wb)
            m1 = jnp.where(t1, m2, m1)
            b1 = jnp.where(t1, b2, b1)
            t2 = t1 | (b2 == wb)
            m2 = jnp.where(t2, NEG, m2)

        w0, w1, w2 = wbs
        lo01 = jnp.minimum(w0, w1)
        hi01 = jnp.maximum(w0, w1)
        srt0 = jnp.minimum(lo01, w2)
        srt2 = jnp.maximum(hi01, w2)
        srt1 = (w0 + w1 + w2) - srt0 - srt2
        ks = [srt0[0], srt1[0], srt2[0]]

        ccps = [pltpu.async_copy(
            in_hbm.at[pl.ds(row0, 8),
                      pl.ds(pl.multiple_of(qcol0 + ks[t] * BCOLS, 128),
                            BCOLS)],
            cbuf.at[pl.ds(t * 8, 8), :], sem0) for t in range(3)]
        m0 = negv
        m1 = negv
        m2 = negv
        i0 = zero
        i1 = zero
        i2 = zero
        for t in range(3):
            ccps[t].wait()
            bvec = qcol0v + ks[t] * BCOLS + lanes

            def scan_body(j, carry, t=t, bvec=bvec):
                m0, m1, m2, i0, i1, i2 = carry
                v = cbuf[t * 8 + sl, pl.ds(j * L, L)]
                vi = bvec + j * L
                v = jnp.where(vi < ROW, v, negv)
                c0 = v > m0
                c1 = v > m1
                c2 = v > m2
                nm2 = jnp.where(c1, m1, jnp.where(c2, v, m2))
                ni2 = jnp.where(c1, i1, jnp.where(c2, vi, i2))
                nm1 = jnp.where(c0, m0, jnp.where(c1, v, m1))
                ni1 = jnp.where(c0, i0, jnp.where(c1, vi, i1))
                nm0 = jnp.where(c0, v, m0)
                ni0 = jnp.where(c0, vi, i0)
                return (nm0, nm1, nm2, ni0, ni1, ni2)

            m0, m1, m2, i0, i1, i2 = lax.fori_loop(
                0, BV, scan_body, (m0, m1, m2, i0, i1, i2))

        slv = zero + sl
        off0 = jnp.where(slv >= 4, 16, zero)
        off1 = jnp.where(slv < 4, 16, zero)
        slotbase = lax.bitwise_and(slv, 3) * 4
        for k in range(3):
            cur = allreduce(m0, jnp.maximum)
            cand = jnp.where(m0 == cur, i0, IMAX)
            widx = allreduce(cand, jnp.minimum)
            isw = (m0 == cur) & (i0 == widx)
            hit0 = lanes == slotbase + k + off0
            hit1 = lanes == slotbase + k + off1
            sv0 = jnp.where(hit0, cur, sv0)
            si0 = jnp.where(hit0, widx, si0)
            sv1 = jnp.where(hit1, cur, sv1)
            si1 = jnp.where(hit1, widx, si1)
            m0 = jnp.where(isw, m1, m0)
            i0 = jnp.where(isw, i1, i0)
            m1 = jnp.where(isw, m2, m1)
            i1 = jnp.where(isw, i2, i1)
            m2 = jnp.where(isw, NEG, m2)
        return (sv0, sv1, si0, si1)

    sv0, sv1, si0, si1 = lax.fori_loop(
        0, 8, row_body, (negv, negv, zero, zero))

    stgv[pl.ds(0, L)] = sv0
    stgv[pl.ds(L, L)] = sv1
    stgi[pl.ds(0, L)] = si0
    stgi[pl.ds(L, L)] = si1
    pltpu.sync_copy(stgv, sval_sh.at[pl.ds(s * 32, 32)])
    pltpu.sync_copy(stgi, sidx_sh.at[pl.ds(s * 32, 32)])
    plsc.subcore_barrier()

    grl = lax.shift_right_logical(s, 2)
    pltpu.sync_copy(sval_sh.at[pl.ds(grl * 128, 128)], mval.at[pl.ds(0, 128)])
    pltpu.sync_copy(sidx_sh.at[pl.ds(grl * 128, 128)], midx.at[pl.ds(0, 128)])
    sl0 = s * 2 - grl * 8
    res = zero
    for p in range(2):
        sl = sl0 + p
        comb_v = negv
        comb_i = zero
        for qq in range(4):
            off = qq * 32 + sl * 4
            vq_v = mval[pl.ds(off, L)]
            vq_i = midx[pl.ds(off, L)]
            perm = jnp.maximum(lanes - 4 * qq, 0)
            sh_v = vq_v.at[perm].get(mode="promise_in_bounds")
            sh_i = vq_i.at[perm].get(mode="promise_in_bounds")
            sel = (lanes >= 4 * qq) & (lanes < 4 * qq + 3)
            comb_v = jnp.where(sel, sh_v, comb_v)
            comb_i = jnp.where(sel, sh_i, comb_i)
        r = c * 32 + s * 2 + p
        base_vec = (zero + r) * cbs_vec
        m0 = comb_v
        i0 = comb_i
        m1 = negv
        m2 = negv
        i1 = zero
        i2 = zero
        for k in range(3):
            cur = allreduce(m0, jnp.maximum)
            cand = jnp.where(m0 == cur, i0, IMAX)
            widx = allreduce(cand, jnp.minimum)
            isw = (m0 == cur) & (i0 == widx)
            qv = (jnp.where(widx >= VOCAB, 1, zero)
                  + jnp.where(widx >= 2 * VOCAB, 1, zero))
            tok = widx - qv * VOCAB
            beam = qv + base_vec
            res = jnp.where(lanes == 6 * p + k, tok, res)
            res = jnp.where(lanes == 6 * p + 3 + k, beam, res)
            m0 = jnp.where(isw, m1, m0)
            i0 = jnp.where(isw, i1, i0)
            m1 = jnp.where(isw, m2, m1)
            i1 = jnp.where(isw, i2, i1)
            m2 = jnp.where(isw, NEG, m2)
    resv[...] = res
    w = c * NS + s
    pltpu.sync_copy(resv, out_hbm.at[pl.ds(w * L, L)])


def kernel(input, index, cur_beam_size):
    cbs = jnp.asarray(cur_beam_size, jnp.int32)
    cbs_arr = jnp.full((L,), cbs, jnp.int32)
    out = _topk_sc(input, cbs_arr)
    x = out.reshape(NC * NS, L)[:, :12].reshape(NC * NS, 2, 2, 3)
    toks = x[:, :, 0, :].reshape(1, BATCH * 3)
    beams = x[:, :, 1, :].reshape(BATCH * 3)
    return toks, beams


# speedup vs baseline: 13.1859x; 1.0563x over previous
import functools

import jax
import jax.numpy as jnp
from jax import lax
from jax.experimental import pallas as pl
from jax.experimental.pallas import tpu as pltpu
from jax.experimental.pallas import tpu_sc as plsc

VOCAB = 100000
BATCH = 64
ROW = 3 * VOCAB
NC, NS, L = 2, 16, 16
NT = 2344
QT = 586
QCOLS = QT * 128
CT = 32
CCOLS = CT * 128
NFC = 18
TAILT = QT - NFC * CT
TCOLS = TAILT * 128
BCOLS = 512
NBQ = 147
BV = BCOLS // L
FS_MAX = QCOLS - BCOLS
NEG = float(jnp.finfo(jnp.float32).min)
IMAX = 2**31 - 1

_mesh = plsc.VectorSubcoreMesh(
    core_axis_name="c", subcore_axis_name="s", num_cores=NC, num_subcores=NS)


@functools.partial(
    pl.kernel,
    out_type=jax.ShapeDtypeStruct((NC * NS * L,), jnp.int32),
    mesh=_mesh,
    compiler_params=pltpu.CompilerParams(use_tc_tiling_on_sc=True),
    scratch_types=[
        pltpu.VMEM((8, CCOLS), jnp.float32),
        pltpu.VMEM((8, CCOLS), jnp.float32),
        pltpu.VMEM((8, TCOLS), jnp.float32),
        pltpu.VMEM((8 * NBQ * L,), jnp.float32),
        pltpu.VMEM((24, BCOLS), jnp.float32),
        pltpu.VMEM((160,), jnp.float32),
        pltpu.VMEM((160,), jnp.int32),
        pltpu.VMEM((32,), jnp.float32),
        pltpu.VMEM((32,), jnp.int32),
        pltpu.VMEM((L,), jnp.int32),
        pltpu.VMEM((L,), jnp.int32),
        pltpu.VMEM_SHARED((NS * 32,), jnp.float32),
        pltpu.VMEM_SHARED((NS * 32,), jnp.int32),
        pltpu.SemaphoreType.DMA,
        pltpu.SemaphoreType.DMA,
    ],
)
def _topk_sc(in_hbm, cbs_hbm, out_hbm, buf0, buf1, tbuf, bmax, cbuf, mval,
             midx, stgv, stgi, cbsv, resv, sval_sh, sidx_sh, sem0, sem1):
    c = lax.axis_index("c")
    s = lax.axis_index("s")
    g = c * 4 + lax.shift_right_logical(s, 2)
    q = lax.bitwise_and(s, 3)
    row0 = pl.multiple_of(g * 8, 8)
    qcol0 = q * QCOLS
    lanes = lax.iota(jnp.int32, L)
    zero = jnp.zeros((L,), jnp.int32)
    negv = jnp.full((L,), NEG, jnp.float32)
    qcol0v = zero + qcol0
    cbs_vec = None
    pltpu.sync_copy(cbs_hbm, cbsv)
    cbs_vec = cbsv[...]
    bufs = (buf0, buf1)
    sems = (sem0, sem1)

    def allreduce(v, op):
        for si in (1, 2, 4, 8):
            perm = jnp.bitwise_xor(lanes, si)
            v = op(v, v.at[perm].get(mode="promise_in_bounds"))
        return v

    def src_slab(start_tile, ncols):
        coff = pl.multiple_of(qcol0 + start_tile * 128, 128)
        return in_hbm.at[pl.ds(row0, 8), pl.ds(coff, ncols)]

    def treemax(vs):
        while len(vs) > 1:
            nxt = [jnp.maximum(vs[t], vs[t + 1])
                   for t in range(0, len(vs) - 1, 2)]
            if len(vs) % 2:
                nxt.append(vs[-1])
            vs = nxt
        return vs[0]

    cps = [pltpu.async_copy(src_slab(0, CCOLS), buf0, sem0)]
    for ci in range(NFC):
        if ci + 1 < NFC:
            cps.append(pltpu.async_copy(
                src_slab((ci + 1) * CT, CCOLS),
                bufs[(ci + 1) % 2], sems[(ci + 1) % 2]))
        elif ci + 1 == NFC:
            cps.append(pltpu.async_copy(
                src_slab(NFC * CT, TCOLS), tbuf, sems[(ci + 1) % 2]))
        cps[ci].wait()
        buf = bufs[ci % 2]
        cblk0 = ci * (CT // 4)

        def blk_body(ii, carry, buf=buf, cblk0=cblk0):
            sl = lax.shift_right_logical(ii, 3)
            b = lax.bitwise_and(ii, 7)
            vs = [buf[sl, pl.ds(b * BCOLS + j * L, L)] for j in range(BV)]
            bmax[pl.ds((sl * NBQ + cblk0 + b) * L, L)] = treemax(vs)
            return carry

        lax.fori_loop(0, 8 * (CT // 4), blk_body, 0)

    cps[NFC].wait()

    def tail_body(ii, carry):
        sl = lax.shift_right_logical(ii, 1)
        b = lax.bitwise_and(ii, 1)
        vs = [tbuf[sl, pl.ds(b * BCOLS + j * L, L)] for j in range(BV)]
        bmax[pl.ds((sl * NBQ + NFC * (CT // 4) + b) * L, L)] = treemax(vs)
        return carry

    lax.fori_loop(0, 16, tail_body, 0)

    def part_body(sl, carry):
        vs = []
        for j in range(BV // 2):
            v = tbuf[sl, pl.ds(2 * BCOLS + j * L, L)]
            colv = qcol0v + (NBQ - 1) * BCOLS + j * L + lanes
            vs.append(jnp.where(colv < ROW, v, negv))
        bmax[pl.ds((sl * NBQ + NBQ - 1) * L, L)] = treemax(vs)
        return carry

    lax.fori_loop(0, 8, part_body, 0)

    def row_body(sl, rcarry):
        sv0, sv1, si0, si1 = rcarry
        m0 = negv
        m1 = negv
        m2 = negv
        b0 = zero
        b1 = zero
        b2 = zero

        def ins_body(blk, carry):
            m0, m1, m2, b0, b1, b2 = carry
            bm = bmax[pl.ds((sl * NBQ + blk) * L, L)]
            bv = zero + blk
            c0 = bm > m0
            c1 = bm > m1
            c2 = bm > m2
            nm2 = jnp.where(c1, m1, jnp.where(c2, bm, m2))
            nb2 = jnp.where(c1, b1, jnp.where(c2, bv, b2))
            nm1 = jnp.where(c0, m0, jnp.where(c1, bm, m1))
            nb1 = jnp.where(c0, b0, jnp.where(c1, bv, b1))
            nm0 = jnp.where(c0, bm, m0)
            nb0 = jnp.where(c0, bv, b0)
            return (nm0, nm1, nm2, nb0, nb1, nb2)

        m0, m1, m2, b0, b1, b2 = lax.fori_loop(
            0, NBQ, ins_body, (m0, m1, m2, b0, b1, b2))

        wbs = []
        for k in range(3):
            cur = allreduce(m0, jnp.maximum)
            cand = jnp.minimum(
                jnp.where(m0 == cur, b0, IMAX),
                jnp.minimum(jnp.where(m1 == cur, b1, IMAX),
                            jnp.where(m2 == cur, b2, IMAX)))
            wb = allreduce(cand, jnp.minimum)
            wbs.append(wb)
            t0 = b0 == wb
            m0 = jnp.where(t0, m1, m0)
            b0 = jnp.where(t0, b1, b0)
            t1 = t0 | (b1 == wb)
            m1 = jnp.where(t1, m2, m1)
            b1 = jnp.where(t1, b2, b1)
            t2 = t1 | (b2 == wb)
            m2 = jnp.where(t2, NEG, m2)

        w0, w1, w2 = wbs
        lo01 = jnp.minimum(w0, w1)
        hi01 = jnp.maximum(w0, w1)
        srt0 = jnp.minimum(lo01, w2)
        srt2 = jnp.maximum(hi01, w2)
        srt1 = (w0 + w1 + w2) - srt0 - srt2
        ks = [srt0[0], srt1[0], srt2[0]]

        fss = [jnp.minimum(ks[t] * BCOLS, FS_MAX) for t in range(3)]
        ccps = [pltpu.async_copy(
            in_hbm.at[pl.ds(row0, 8),
                      pl.ds(pl.multiple_of(qcol0 + fss[t], 128), BCOLS)],
            cbuf.at[pl.ds(t * 8, 8), :], sem0) for t in range(3)]
        m0 = negv
        m1 = negv
        m2 = negv
        i0 = zero
        i1 = zero
        i2 = zero
        for t in range(3):
            ccps[t].wait()
            bvec = qcol0v + fss[t] + lanes
            lov = qcol0v + ks[t] * BCOLS
            hiv = jnp.minimum(jnp.minimum(lov + BCOLS, qcol0v + QCOLS),
                              zero + ROW)

            def scan_body(j, carry, t=t, bvec=bvec, lov=lov, hiv=hiv):
                m0, m1, m2, i0, i1, i2 = carry
                v = cbuf[t * 8 + sl, pl.ds(j * L, L)]
                vi = bvec + j * L
                v = jnp.where(vi >= lov, v, negv)
                v = jnp.where(vi < hiv, v, negv)
                c0 = v > m0
                c1 = v > m1
                c2 = v > m2
                nm2 = jnp.where(c1, m1, jnp.where(c2, v, m2))
                ni2 = jnp.where(c1, i1, jnp.where(c2, vi, i2))
                nm1 = jnp.where(c0, m0, jnp.where(c1, v, m1))
                ni1 = jnp.where(c0, i0, jnp.where(c1, vi, i1))
                nm0 = jnp.where(c0, v, m0)
                ni0 = jnp.where(c0, vi, i0)
                return (nm0, nm1, nm2, ni0, ni1, ni2)

            m0, m1, m2, i0, i1, i2 = lax.fori_loop(
                0, BV, scan_body, (m0, m1, m2, i0, i1, i2))

        slv = zero + sl
        off0 = jnp.where(slv >= 4, 16, zero)
        off1 = jnp.where(slv < 4, 16, zero)
        slotbase = lax.bitwise_and(slv, 3) * 4
        for k in range(3):
            cur = allreduce(m0, jnp.maximum)
            cand = jnp.where(m0 == cur, i0, IMAX)
            widx = allreduce(cand, jnp.minimum)
            isw = (m0 == cur) & (i0 == widx)
            hit0 = lanes == slotbase + k + off0
            hit1 = lanes == slotbase + k + off1
            sv0 = jnp.where(hit0, cur, sv0)
            si0 = jnp.where(hit0, widx, si0)
            sv1 = jnp.where(hit1, cur, sv1)
            si1 = jnp.where(hit1, widx, si1)
            m0 = jnp.where(isw, m1, m0)
            i0 = jnp.where(isw, i1, i0)
            m1 = jnp.where(isw, m2, m1)
            i1 = jnp.where(isw, i2, i1)
            m2 = jnp.where(isw, NEG, m2)
        return (sv0, sv1, si0, si1)

    sv0, sv1, si0, si1 = lax.fori_loop(
        0, 8, row_body, (negv, negv, zero, zero))

    stgv[pl.ds(0, L)] = sv0
    stgv[pl.ds(L, L)] = sv1
    stgi[pl.ds(0, L)] = si0
    stgi[pl.ds(L, L)] = si1
    pltpu.sync_copy(stgv, sval_sh.at[pl.ds(s * 32, 32)])
    pltpu.sync_copy(stgi, sidx_sh.at[pl.ds(s * 32, 32)])
    plsc.subcore_barrier()

    grl = lax.shift_right_logical(s, 2)
    pltpu.sync_copy(sval_sh.at[pl.ds(grl * 128, 128)], mval.at[pl.ds(0, 128)])
    pltpu.sync_copy(sidx_sh.at[pl.ds(grl * 128, 128)], midx.at[pl.ds(0, 128)])
    sl0 = s * 2 - grl * 8
    res = zero
    for p in range(2):
        sl = sl0 + p
        comb_v = negv
        comb_i = zero
        for qq in range(4):
            off = qq * 32 + sl * 4
            vq_v = mval[pl.ds(off, L)]
            vq_i = midx[pl.ds(off, L)]
            perm = jnp.maximum(lanes - 4 * qq, 0)
            sh_v = vq_v.at[perm].get(mode="promise_in_bounds")
            sh_i = vq_i.at[perm].get(mode="promise_in_bounds")
            sel = (lanes >= 4 * qq) & (lanes < 4 * qq + 3)
            comb_v = jnp.where(sel, sh_v, comb_v)
            comb_i = jnp.where(sel, sh_i, comb_i)
        r = c * 32 + s * 2 + p
        base_vec = (zero + r) * cbs_vec
        m0 = comb_v
        i0 = comb_i
        m1 = negv
        m2 = negv
        i1 = zero
        i2 = zero
        for k in range(3):
            cur = allreduce(m0, jnp.maximum)
            cand = jnp.where(m0 == cur, i0, IMAX)
            widx = allreduce(cand, jnp.minimum)
            isw = (m0 == cur) & (i0 == widx)
            qv = (jnp.where(widx >= VOCAB, 1, zero)
                  + jnp.where(widx >= 2 * VOCAB, 1, zero))
            tok = widx - qv * VOCAB
            beam = qv + base_vec
            res = jnp.where(lanes == 6 * p + k, tok, res)
            res = jnp.where(lanes == 6 * p + 3 + k, beam, res)
            m0 = jnp.where(isw, m1, m0)
            i0 = jnp.where(isw, i1, i0)
            m1 = jnp.where(isw, m2, m1)
            i1 = jnp.where(isw, i2, i1)
            m2 = jnp.where(isw, NEG, m2)
    resv[...] = res
    w = c * NS + s
    pltpu.sync_copy(resv, out_hbm.at[pl.ds(w * L, L)])


def kernel(input, index, cur_beam_size):
    cbs = jnp.asarray(cur_beam_size, jnp.int32)
    cbs_arr = jnp.full((L,), cbs, jnp.int32)
    out = _topk_sc(input, cbs_arr)
    x = out.reshape(NC * NS, L)[:, :12].reshape(NC * NS, 2, 2, 3)
    toks = x[:, :, 0, :].reshape(1, BATCH * 3)
    beams = x[:, :, 1, :].reshape(BATCH * 3)
    return toks, beams


# speedup vs baseline: 13.7142x; 1.0401x over previous
import functools

import jax
import jax.numpy as jnp
from jax import lax
from jax.experimental import pallas as pl
from jax.experimental.pallas import tpu as pltpu
from jax.experimental.pallas import tpu_sc as plsc

VOCAB = 100000
BATCH = 64
ROW = 3 * VOCAB
NC, NS, L = 2, 16, 16
NT = 2344
QT = 586
QCOLS = QT * 128
CT = 32
CCOLS = CT * 128
NFC = 18
TAILT = QT - NFC * CT
TCOLS = TAILT * 128
BCOLS = 512
NBQ = 147
BV = BCOLS // L
FS_MAX = QCOLS - BCOLS
NEG = float(jnp.finfo(jnp.float32).min)
IMAX = 2**31 - 1

_mesh = plsc.VectorSubcoreMesh(
    core_axis_name="c", subcore_axis_name="s", num_cores=NC, num_subcores=NS)


@functools.partial(
    pl.kernel,
    out_type=jax.ShapeDtypeStruct((NC * NS * L,), jnp.int32),
    mesh=_mesh,
    compiler_params=pltpu.CompilerParams(use_tc_tiling_on_sc=True),
    scratch_types=[
        pltpu.VMEM((8, CCOLS), jnp.float32),
        pltpu.VMEM((8, CCOLS), jnp.float32),
        pltpu.VMEM((8, TCOLS), jnp.float32),
        pltpu.VMEM((8 * NBQ * L,), jnp.float32),
        pltpu.VMEM((48, BCOLS), jnp.float32),
        pltpu.VMEM((24 * L,), jnp.int32),
        pltpu.VMEM((160,), jnp.float32),
        pltpu.VMEM((160,), jnp.int32),
        pltpu.VMEM((32,), jnp.float32),
        pltpu.VMEM((32,), jnp.int32),
        pltpu.VMEM((L,), jnp.int32),
        pltpu.VMEM((L,), jnp.int32),
        pltpu.VMEM_SHARED((NS * 32,), jnp.float32),
        pltpu.VMEM_SHARED((NS * 32,), jnp.int32),
        pltpu.SemaphoreType.DMA,
        pltpu.SemaphoreType.DMA,
    ],
)
def _topk_sc(in_hbm, cbs_hbm, out_hbm, buf0, buf1, tbuf, bmax, cbuf, wblk,
             mval, midx, stgv, stgi, cbsv, resv, sval_sh, sidx_sh,
             sem0, sem1):
    c = lax.axis_index("c")
    s = lax.axis_index("s")
    g = c * 4 + lax.shift_right_logical(s, 2)
    q = lax.bitwise_and(s, 3)
    row0 = pl.multiple_of(g * 8, 8)
    qcol0 = q * QCOLS
    lanes = lax.iota(jnp.int32, L)
    zero = jnp.zeros((L,), jnp.int32)
    negv = jnp.full((L,), NEG, jnp.float32)
    qcol0v = zero + qcol0
    cbs_vec = None
    pltpu.sync_copy(cbs_hbm, cbsv)
    cbs_vec = cbsv[...]
    bufs = (buf0, buf1)
    sems = (sem0, sem1)

    def allreduce(v, op):
        for si in (1, 2, 4, 8):
            perm = jnp.bitwise_xor(lanes, si)
            v = op(v, v.at[perm].get(mode="promise_in_bounds"))
        return v

    def src_slab(start_tile, ncols):
        coff = pl.multiple_of(qcol0 + start_tile * 128, 128)
        return in_hbm.at[pl.ds(row0, 8), pl.ds(coff, ncols)]

    def treemax(vs):
        while len(vs) > 1:
            nxt = [jnp.maximum(vs[t], vs[t + 1])
                   for t in range(0, len(vs) - 1, 2)]
            if len(vs) % 2:
                nxt.append(vs[-1])
            vs = nxt
        return vs[0]

    cps = [pltpu.async_copy(src_slab(0, CCOLS), buf0, sem0)]
    for ci in range(NFC):
        if ci + 1 < NFC:
            cps.append(pltpu.async_copy(
                src_slab((ci + 1) * CT, CCOLS),
                bufs[(ci + 1) % 2], sems[(ci + 1) % 2]))
        elif ci + 1 == NFC:
            cps.append(pltpu.async_copy(
                src_slab(NFC * CT, TCOLS), tbuf, sems[(ci + 1) % 2]))
        cps[ci].wait()
        buf = bufs[ci % 2]
        cblk0 = ci * (CT // 4)

        def blk_body(ii, carry, buf=buf, cblk0=cblk0):
            sl = lax.shift_right_logical(ii, 3)
            b = lax.bitwise_and(ii, 7)
            vs = [buf[sl, pl.ds(b * BCOLS + j * L, L)] for j in range(BV)]
            bmax[pl.ds((sl * NBQ + cblk0 + b) * L, L)] = treemax(vs)
            return carry

        lax.fori_loop(0, 8 * (CT // 4), blk_body, 0)

    cps[NFC].wait()

    def tail_body(ii, carry):
        sl = lax.shift_right_logical(ii, 1)
        b = lax.bitwise_and(ii, 1)
        vs = [tbuf[sl, pl.ds(b * BCOLS + j * L, L)] for j in range(BV)]
        bmax[pl.ds((sl * NBQ + NFC * (CT // 4) + b) * L, L)] = treemax(vs)
        return carry

    lax.fori_loop(0, 16, tail_body, 0)

    def part_body(sl, carry):
        vs = []
        for j in range(BV // 2):
            v = tbuf[sl, pl.ds(2 * BCOLS + j * L, L)]
            colv = qcol0v + (NBQ - 1) * BCOLS + j * L + lanes
            vs.append(jnp.where(colv < ROW, v, negv))
        bmax[pl.ds((sl * NBQ + NBQ - 1) * L, L)] = treemax(vs)
        return carry

    lax.fori_loop(0, 8, part_body, 0)

    def row_body(sl, rcarry):
        m0 = negv
        m1 = negv
        m2 = negv
        b0 = zero
        b1 = zero
        b2 = zero

        def ins_body(blk, carry):
            m0, m1, m2, b0, b1, b2 = carry
            bm = bmax[pl.ds((sl * NBQ + blk) * L, L)]
            bv = zero + blk
            c0 = bm > m0
            c1 = bm > m1
            c2 = bm > m2
            nm2 = jnp.where(c1, m1, jnp.where(c2, bm, m2))
            nb2 = jnp.where(c1, b1, jnp.where(c2, bv, b2))
            nm1 = jnp.where(c0, m0, jnp.where(c1, bm, m1))
            nb1 = jnp.where(c0, b0, jnp.where(c1, bv, b1))
            nm0 = jnp.where(c0, bm, m0)
            nb0 = jnp.where(c0, bv, b0)
            return (nm0, nm1, nm2, nb0, nb1, nb2)

        m0, m1, m2, b0, b1, b2 = lax.fori_loop(
            0, NBQ, ins_body, (m0, m1, m2, b0, b1, b2))

        wbs = []
        for k in range(3):
            cur = allreduce(m0, jnp.maximum)
            cand = jnp.minimum(
                jnp.where(m0 == cur, b0, IMAX),
                jnp.minimum(jnp.where(m1 == cur, b1, IMAX),
                            jnp.where(m2 == cur, b2, IMAX)))
            wb = allreduce(cand, jnp.minimum)
            wbs.append(wb)
            t0 = b0 == wb
            m0 = jnp.where(t0, m1, m0)
            b0 = jnp.where(t0, b1, b0)
            t1 = t0 | (b1 == wb)
            m1 = jnp.where(t1, m2, m1)
            b1 = jnp.where(t1, b2, b1)
            t2 = t1 | (b2 == wb)
            m2 = jnp.where(t2, NEG, m2)

        w0, w1, w2 = wbs
        lo01 = jnp.minimum(w0, w1)
        hi01 = jnp.maximum(w0, w1)
        srt0 = jnp.minimum(lo01, w2)
        srt2 = jnp.maximum(hi01, w2)
        srt1 = (w0 + w1 + w2) - srt0 - srt2
        wblk[pl.ds((sl * 3 + 0) * L, L)] = srt0
        wblk[pl.ds((sl * 3 + 1) * L, L)] = srt1
        wblk[pl.ds((sl * 3 + 2) * L, L)] = srt2
        return rcarry

    lax.fori_loop(0, 8, row_body, 0)

    def cdma(sl, t, half):
        k = wblk[pl.ds((sl * 3 + t) * L, L)][0]
        fs = jnp.minimum(k * BCOLS, FS_MAX)
        src = in_hbm.at[pl.ds(row0, 8),
                        pl.ds(pl.multiple_of(qcol0 + fs, 128), BCOLS)]
        dst = cbuf.at[pl.ds(half * 24 + t * 8, 8), :]
        return k, fs, src, dst

    for par in range(2):
        for t in range(3):
            _, _, src, dst = cdma(par, t, par)
            pltpu.async_copy(src, dst, sems[par])

    def duo_body(d, rcarry):
        sv0, sv1, si0, si1 = rcarry
        for par in range(2):
            sl = d * 2 + par
            sem = sems[par]
            kss = []
            fsss = []
            for t in range(3):
                k, fs, src, dst = cdma(sl, t, par)
                pltpu.make_async_copy(src, dst, sem).wait()
                kss.append(k)
                fsss.append(fs)
            m0 = negv
            m1 = negv
            m2 = negv
            i0 = zero
            i1 = zero
            i2 = zero
            for t in range(3):
                bvec = qcol0v + fsss[t] + lanes
                lov = qcol0v + kss[t] * BCOLS
                hiv = jnp.minimum(jnp.minimum(lov + BCOLS, qcol0v + QCOLS),
                                  zero + ROW)

                def scan_body(j, carry, t=t, par=par, sl=sl, bvec=bvec,
                              lov=lov, hiv=hiv):
                    m0, m1, m2, i0, i1, i2 = carry
                    v = cbuf[par * 24 + t * 8 + sl, pl.ds(j * L, L)]
                    vi = bvec + j * L
                    v = jnp.where(vi >= lov, v, negv)
                    v = jnp.where(vi < hiv, v, negv)
                    c0 = v > m0
                    c1 = v > m1
                    c2 = v > m2
                    nm2 = jnp.where(c1, m1, jnp.where(c2, v, m2))
                    ni2 = jnp.where(c1, i1, jnp.where(c2, vi, i2))
                    nm1 = jnp.where(c0, m0, jnp.where(c1, v, m1))
                    ni1 = jnp.where(c0, i0, jnp.where(c1, vi, i1))
                    nm0 = jnp.where(c0, v, m0)
                    ni0 = jnp.where(c0, vi, i0)
                    return (nm0, nm1, nm2, ni0, ni1, ni2)

                m0, m1, m2, i0, i1, i2 = lax.fori_loop(
                    0, BV, scan_body, (m0, m1, m2, i0, i1, i2))

            @pl.when(d < 3)
            def _issue_next(sl=sl, par=par, sem=sem):
                for t in range(3):
                    _, _, src, dst = cdma(sl + 2, t, par)
                    pltpu.async_copy(src, dst, sem)

            slv = zero + sl
            off0 = jnp.where(slv >= 4, 16, zero)
            off1 = jnp.where(slv < 4, 16, zero)
            slotbase = lax.bitwise_and(slv, 3) * 4
            for k in range(3):
                cur = allreduce(m0, jnp.maximum)
                cand = jnp.where(m0 == cur, i0, IMAX)
                widx = allreduce(cand, jnp.minimum)
                isw = (m0 == cur) & (i0 == widx)
                hit0 = lanes == slotbase + k + off0
                hit1 = lanes == slotbase + k + off1
                sv0 = jnp.where(hit0, cur, sv0)
                si0 = jnp.where(hit0, widx, si0)
                sv1 = jnp.where(hit1, cur, sv1)
                si1 = jnp.where(hit1, widx, si1)
                m0 = jnp.where(isw, m1, m0)
                i0 = jnp.where(isw, i1, i0)
                m1 = jnp.where(isw, m2, m1)
                i1 = jnp.where(isw, i2, i1)
                m2 = jnp.where(isw, NEG, m2)
        return (sv0, sv1, si0, si1)

    sv0, sv1, si0, si1 = lax.fori_loop(
        0, 4, duo_body, (negv, negv, zero, zero))

    stgv[pl.ds(0, L)] = sv0
    stgv[pl.ds(L, L)] = sv1
    stgi[pl.ds(0, L)] = si0
    stgi[pl.ds(L, L)] = si1
    pltpu.sync_copy(stgv, sval_sh.at[pl.ds(s * 32, 32)])
    pltpu.sync_copy(stgi, sidx_sh.at[pl.ds(s * 32, 32)])
    plsc.subcore_barrier()

    grl = lax.shift_right_logical(s, 2)
    pltpu.sync_copy(sval_sh.at[pl.ds(grl * 128, 128)], mval.at[pl.ds(0, 128)])
    pltpu.sync_copy(sidx_sh.at[pl.ds(grl * 128, 128)], midx.at[pl.ds(0, 128)])
    sl0 = s * 2 - grl * 8
    res = zero
    for p in range(2):
        sl = sl0 + p
        comb_v = negv
        comb_i = zero
        for qq in range(4):
            off = qq * 32 + sl * 4
            vq_v = mval[pl.ds(off, L)]
            vq_i = midx[pl.ds(off, L)]
            perm = jnp.maximum(lanes - 4 * qq, 0)
            sh_v = vq_v.at[perm].get(mode="promise_in_bounds")
            sh_i = vq_i.at[perm].get(mode="promise_in_bounds")
            sel = (lanes >= 4 * qq) & (lanes < 4 * qq + 3)
            comb_v = jnp.where(sel, sh_v, comb_v)
            comb_i = jnp.where(sel, sh_i, comb_i)
        r = c * 32 + s * 2 + p
        base_vec = (zero + r) * cbs_vec
        m0 = comb_v
        i0 = comb_i
        m1 = negv
        m2 = negv
        i1 = zero
        i2 = zero
        for k in range(3):
            cur = allreduce(m0, jnp.maximum)
            cand = jnp.where(m0 == cur, i0, IMAX)
            widx = allreduce(cand, jnp.minimum)
            isw = (m0 == cur) & (i0 == widx)
            qv = (jnp.where(widx >= VOCAB, 1, zero)
                  + jnp.where(widx >= 2 * VOCAB, 1, zero))
            tok = widx - qv * VOCAB
            beam = qv + base_vec
            res = jnp.where(lanes == 6 * p + k, tok, res)
            res = jnp.where(lanes == 6 * p + 3 + k, beam, res)
            m0 = jnp.where(isw, m1, m0)
            i0 = jnp.where(isw, i1, i0)
            m1 = jnp.where(isw, m2, m1)
            i1 = jnp.where(isw, i2, i1)
            m2 = jnp.where(isw, NEG, m2)
    resv[...] = res
    w = c * NS + s
    pltpu.sync_copy(resv, out_hbm.at[pl.ds(w * L, L)])


def kernel(input, index, cur_beam_size):
    cbs = jnp.asarray(cur_beam_size, jnp.int32)
    cbs_arr = jnp.full((L,), cbs, jnp.int32)
    out = _topk_sc(input, cbs_arr)
    x = out.reshape(NC * NS, L)[:, :12].reshape(NC * NS, 2, 2, 3)
    toks = x[:, :, 0, :].reshape(1, BATCH * 3)
    beams = x[:, :, 1, :].reshape(BATCH * 3)
    return toks, beams
